# baseline (speedup 1.0000x reference)
import functools

import jax
import jax.numpy as jnp
from jax import lax
from jax.experimental import pallas as pl
from jax.experimental.pallas import tpu as pltpu
from jax.experimental.pallas import tpu_sc as plsc

_MAX_STEPS = 1000
_N_ROWS = _MAX_STEPS + 1
_PAD_ROWS = 1008
_D_IN = 128
_D_OUT = 512


def _build_table():
    steps = jnp.arange(_PAD_ROWS, dtype=jnp.float32)[:, None]
    dims = jnp.arange(64, dtype=jnp.float32)[None, :]
    t = steps * (10.0 ** (dims * 4.0 / 63.0))
    return jnp.stack([jnp.cos(t), jnp.sin(t)], axis=-1).reshape(_PAD_ROWS, -1)


_DN = (((1,), (1,)), ((), ()))


def _mlp_body(tab_ref, w1_ref, b1_ref, w2_ref, b2_ref, out_ref):
    h = lax.dot_general(tab_ref[...], w1_ref[...], _DN,
                        preferred_element_type=jnp.float32)
    h = h + b1_ref[...][None, :]
    h = h * jax.nn.sigmoid(h)
    z = lax.dot_general(h, w2_ref[...], _DN,
                        preferred_element_type=jnp.float32)
    z = z + b2_ref[...][None, :]
    out_ref[...] = z * jax.nn.sigmoid(z)


_N_COL_BLK = 2


def _activated_table(table, W1, b1, W2, b2):
    blk = _D_OUT // _N_COL_BLK
    return pl.pallas_call(
        _mlp_body,
        grid=(_N_COL_BLK,),
        in_specs=[
            pl.BlockSpec((_PAD_ROWS, _D_IN), lambda j: (0, 0)),
            pl.BlockSpec((_D_OUT, _D_IN), lambda j: (0, 0)),
            pl.BlockSpec((_D_OUT,), lambda j: (0,)),
            pl.BlockSpec((blk, _D_OUT), lambda j: (j, 0)),
            pl.BlockSpec((blk,), lambda j: (j,)),
        ],
        out_specs=pl.BlockSpec((_PAD_ROWS, blk), lambda j: (0, j)),
        out_shape=jax.ShapeDtypeStruct((_PAD_ROWS, _D_OUT), jnp.float32),
    )(table, W1, b1, W2, b2)


@functools.cache
def _make_gather(batch, d):
    info = plsc.get_sparse_core_info()
    nc, ns = info.num_cores, info.num_subcores
    nw = nc * ns
    b_per_w = batch // nw
    ch = 64
    n_ch = b_per_w // ch
    nbuf = 2
    n_grp = n_ch // nbuf
    mesh = plsc.VectorSubcoreMesh(core_axis_name="c", subcore_axis_name="s")

    @functools.partial(
        pl.kernel,
        mesh=mesh,
        out_type=jax.ShapeDtypeStruct((batch, d), jnp.float32),
        scratch_types=[
            pltpu.VMEM((n_ch, ch), jnp.int32),
        ]
        + [pltpu.VMEM((ch, d), jnp.float32)] * nbuf
        + [pltpu.SemaphoreType.DMA] * (2 * nbuf),
    )
    def gather_rows(tab_hbm, idx_hbm, out_hbm, idx_v, *bufs_sems):
        bufs = bufs_sems[:nbuf]
        gsems = bufs_sems[nbuf:2 * nbuf]
        ssems = bufs_sems[2 * nbuf:]
        wid = lax.axis_index("s") * nc + lax.axis_index("c")
        base = wid * b_per_w
        pltpu.sync_copy(idx_hbm.at[wid], idx_v)

        def gather_cp(c, b):
            return pltpu.make_async_copy(
                tab_hbm.at[idx_v.at[c]], bufs[b], gsems[b])

        def store_cp(c, b):
            return pltpu.make_async_copy(
                bufs[b], out_hbm.at[pl.ds(base + c * ch, ch)], ssems[b])

        for b in range(nbuf):
            gather_cp(b, b).start()

        @pl.loop(0, n_grp)
        def _(grp):
            for b in range(nbuf):
                c = grp * nbuf + b
                gather_cp(c, b).wait()
                store_cp(c, b).start()

                @pl.when(grp < n_grp - 1)
                def _():
                    store_cp(c, b).wait()
                    gather_cp(c + nbuf, b).start()

        for b in range(nbuf):
            store_cp(n_ch - nbuf + b, b).wait()

    return gather_rows


def kernel(diffusion_step, W1, b1, W2, b2):
    table = _build_table()
    z = _activated_table(table, W1, b1, W2, b2)
    idx = jnp.clip(diffusion_step, 0, _MAX_STEPS).astype(jnp.int32)
    batch = diffusion_step.shape[0]
    idx3 = idx.reshape(32, batch // (32 * 64), 64)
    return _make_gather(batch, _D_OUT)(z, idx3)


# speedup vs baseline: 1.0100x; 1.0100x over previous
import functools

import jax
import jax.numpy as jnp
from jax import lax
from jax.experimental import pallas as pl
from jax.experimental.pallas import tpu as pltpu
from jax.experimental.pallas import tpu_sc as plsc

_MAX_STEPS = 1000
_N_ROWS = _MAX_STEPS + 1
_PAD_ROWS = 1008
_D_IN = 128
_D_OUT = 512


def _build_table():
    steps = jnp.arange(_PAD_ROWS, dtype=jnp.float32)[:, None]
    dims = jnp.arange(64, dtype=jnp.float32)[None, :]
    t = steps * (10.0 ** (dims * 4.0 / 63.0))
    return jnp.stack([jnp.cos(t), jnp.sin(t)], axis=-1).reshape(_PAD_ROWS, -1)


_DN = (((1,), (1,)), ((), ()))


def _mlp_body(tab_ref, w1_ref, b1_ref, w2_ref, b2_ref, out_ref):
    h = lax.dot_general(tab_ref[...], w1_ref[...], _DN,
                        preferred_element_type=jnp.float32)
    h = h + b1_ref[...][None, :]
    h = h * jax.nn.sigmoid(h)
    z = lax.dot_general(h, w2_ref[...], _DN,
                        preferred_element_type=jnp.float32)
    z = z + b2_ref[...][None, :]
    out_ref[...] = z * jax.nn.sigmoid(z)


def _activated_table(table, W1, b1, W2, b2):
    return pl.pallas_call(
        _mlp_body,
        out_shape=jax.ShapeDtypeStruct((_PAD_ROWS, _D_OUT), jnp.float32),
    )(table, W1, b1, W2, b2)


@functools.cache
def _make_gather(batch, d):
    info = plsc.get_sparse_core_info()
    nc, ns = info.num_cores, info.num_subcores
    nw = nc * ns
    b_per_w = batch // nw
    ch = 64
    n_ch = b_per_w // ch
    nbuf = 2
    n_grp = n_ch // nbuf
    mesh = plsc.VectorSubcoreMesh(core_axis_name="c", subcore_axis_name="s")

    @functools.partial(
        pl.kernel,
        mesh=mesh,
        out_type=jax.ShapeDtypeStruct((batch, d), jnp.float32),
        scratch_types=[
            pltpu.VMEM((n_ch, ch), jnp.int32),
        ]
        + [pltpu.VMEM((ch, d), jnp.float32)] * nbuf
        + [pltpu.SemaphoreType.DMA] * (2 * nbuf),
    )
    def gather_rows(tab_hbm, idx_hbm, out_hbm, idx_v, *bufs_sems):
        bufs = bufs_sems[:nbuf]
        gsems = bufs_sems[nbuf:2 * nbuf]
        ssems = bufs_sems[2 * nbuf:]
        wid = lax.axis_index("s") * nc + lax.axis_index("c")
        base = wid * b_per_w
        pltpu.sync_copy(idx_hbm.at[wid], idx_v)

        def gather_cp(c, b):
            return pltpu.make_async_copy(
                tab_hbm.at[idx_v.at[c]], bufs[b], gsems[b])

        def store_cp(c, b):
            return pltpu.make_async_copy(
                bufs[b], out_hbm.at[pl.ds(base + c * ch, ch)], ssems[b])

        for b in range(nbuf):
            gather_cp(b, b).start()

        @pl.loop(0, n_grp)
        def _(grp):
            for b in range(nbuf):
                c = grp * nbuf + b
                gather_cp(c, b).wait()
                store_cp(c, b).start()

                @pl.when(grp < n_grp - 1)
                def _():
                    store_cp(c, b).wait()
                    gather_cp(c + nbuf, b).start()

        for b in range(nbuf):
            store_cp(n_ch - nbuf + b, b).wait()

    return gather_rows


def kernel(diffusion_step, W1, b1, W2, b2):
    table = _build_table()
    z = _activated_table(table, W1, b1, W2, b2)
    idx = jnp.clip(diffusion_step, 0, _MAX_STEPS).astype(jnp.int32)
    batch = diffusion_step.shape[0]
    idx3 = idx.reshape(32, batch // (32 * 64), 64)
    return _make_gather(batch, _D_OUT)(z, idx3)


# speedup vs baseline: 1.0153x; 1.0053x over previous
import functools

import jax
import jax.numpy as jnp
from jax import lax
from jax.experimental import pallas as pl
from jax.experimental.pallas import tpu as pltpu
from jax.experimental.pallas import tpu_sc as plsc

_MAX_STEPS = 1000
_N_ROWS = _MAX_STEPS + 1
_PAD_ROWS = 1008
_D_IN = 128
_D_OUT = 512


def _build_table():
    steps = jnp.arange(_PAD_ROWS, dtype=jnp.float32)[:, None]
    dims = jnp.arange(64, dtype=jnp.float32)[None, :]
    t = steps * (10.0 ** (dims * 4.0 / 63.0))
    return jnp.stack([jnp.cos(t), jnp.sin(t)], axis=-1).reshape(_PAD_ROWS, -1)


_DN = (((1,), (1,)), ((), ()))


def _mlp_body(tab_ref, w1_ref, b1_ref, w2_ref, b2_ref, out_ref):
    h = lax.dot_general(tab_ref[...], w1_ref[...], _DN,
                        preferred_element_type=jnp.float32)
    h = h + b1_ref[...][None, :]
    h = h * jax.nn.sigmoid(h)
    z = lax.dot_general(h, w2_ref[...], _DN,
                        preferred_element_type=jnp.float32)
    z = z + b2_ref[...][None, :]
    out_ref[...] = z * jax.nn.sigmoid(z)


def _activated_table(table, W1, b1, W2, b2):
    return pl.pallas_call(
        _mlp_body,
        out_shape=jax.ShapeDtypeStruct((_PAD_ROWS, _D_OUT), jnp.float32),
    )(table, W1, b1, W2, b2)


@functools.cache
def _make_gather(batch, d):
    info = plsc.get_sparse_core_info()
    nc, ns = info.num_cores, info.num_subcores
    nw = nc * ns
    b_per_w = batch // nw
    ch = 64
    n_ch = b_per_w // ch
    nbuf = 2
    n_grp = n_ch // nbuf
    mesh = plsc.VectorSubcoreMesh(core_axis_name="c", subcore_axis_name="s")

    @functools.partial(
        pl.kernel,
        mesh=mesh,
        out_type=jax.ShapeDtypeStruct((batch, d), jnp.float32),
        scratch_types=[
            pltpu.VMEM((b_per_w,), jnp.int32),
        ]
        + [pltpu.VMEM((ch, d), jnp.float32)] * nbuf
        + [pltpu.SemaphoreType.DMA] * (2 * nbuf),
    )
    def gather_rows(tab_hbm, idx_hbm, out_hbm, idx_v, *bufs_sems):
        bufs = bufs_sems[:nbuf]
        gsems = bufs_sems[nbuf:2 * nbuf]
        ssems = bufs_sems[2 * nbuf:]
        wid = lax.axis_index("s") * nc + lax.axis_index("c")
        base = wid * b_per_w
        pltpu.sync_copy(idx_hbm.at[pl.ds(base, b_per_w)], idx_v)

        def gather_cp(c, b):
            return pltpu.make_async_copy(
                tab_hbm.at[idx_v.at[pl.ds(c * ch, ch)]], bufs[b], gsems[b])

        def store_cp(c, b):
            return pltpu.make_async_copy(
                bufs[b], out_hbm.at[pl.ds(base + c * ch, ch)], ssems[b])

        for b in range(nbuf):
            gather_cp(b, b).start()

        @pl.loop(0, n_grp)
        def _(grp):
            for b in range(nbuf):
                c = grp * nbuf + b
                gather_cp(c, b).wait()
                store_cp(c, b).start()

                @pl.when(grp < n_grp - 1)
                def _():
                    store_cp(c, b).wait()
                    gather_cp(c + nbuf, b).start()

        for b in range(nbuf):
            store_cp(n_ch - nbuf + b, b).wait()

    return gather_rows


def kernel(diffusion_step, W1, b1, W2, b2):
    table = _build_table()
    z = _activated_table(table, W1, b1, W2, b2)
    idx = jnp.clip(diffusion_step, 0, _MAX_STEPS).astype(jnp.int32)
    return _make_gather(diffusion_step.shape[0], _D_OUT)(z, idx)
